# baseline (speedup 1.0000x reference)
import functools

import jax
import jax.numpy as jnp
from jax.experimental import pallas as pl
from jax.experimental.pallas import tpu as pltpu

TOP_K = 2
LANES = 128


def _router_body(nexp, x_ref, g_ref, wout_ref, iout_ref):
    bt = x_ref.shape[0]
    logits = jax.lax.dot_general(
        x_ref[...], g_ref[...], (((1,), (1,)), ((), ())),
        preferred_element_type=jnp.float32)
    io = jax.lax.broadcasted_iota(jnp.int32, (bt, LANES), 1)
    valid = io < nexp
    logits = jnp.where(valid, logits, -jnp.inf)
    m = jnp.max(logits, axis=1, keepdims=True)
    p = jnp.exp(logits - m)
    p = p / jnp.sum(p, axis=1, keepdims=True)
    m1 = jnp.max(p, axis=1, keepdims=True)
    i1 = jnp.min(jnp.where(p == m1, io, LANES), axis=1, keepdims=True)
    p2 = jnp.where(io == i1, -1.0, p)
    m2 = jnp.max(p2, axis=1, keepdims=True)
    i2 = jnp.min(jnp.where(p2 == m2, io, LANES), axis=1, keepdims=True)
    s = m1 + m2
    wa = m1 / s
    wb = m2 / s
    wout_ref[...] = jnp.where(io == 0, wa, jnp.where(io == 1, wb, 0.0))
    iout_ref[...] = jnp.where(io == 0, i1, jnp.where(io == 1, i2, 0))


def _router(x, gate_w, bt=256):
    t, d = x.shape
    e = gate_w.shape[0]
    gate_pad = jnp.zeros((LANES, d), jnp.float32).at[:e].set(gate_w)
    wout, iout = pl.pallas_call(
        functools.partial(_router_body, e),
        grid=(t // bt,),
        in_specs=[
            pl.BlockSpec((bt, d), lambda i: (i, 0)),
            pl.BlockSpec((LANES, d), lambda i: (0, 0)),
        ],
        out_specs=[
            pl.BlockSpec((bt, LANES), lambda i: (i, 0)),
            pl.BlockSpec((bt, LANES), lambda i: (i, 0)),
        ],
        out_shape=[
            jax.ShapeDtypeStruct((t, LANES), jnp.float32),
            jax.ShapeDtypeStruct((t, LANES), jnp.int32),
        ],
    )(x, gate_pad)
    return wout[:, :TOP_K], iout[:, :TOP_K]


def _ffn_body(nf, t, be_ref, x_ref, w1_ref, w3_ref, w2_ref, tok_ref, pw_ref,
              out_ref, xg_ref, y_ref):
    b = pl.program_id(0)
    f = pl.program_id(1)
    bt = xg_ref.shape[0]

    @pl.when(jnp.logical_and(b == 0, f == 0))
    def _init():
        out_ref[...] = jnp.zeros_like(out_ref)

    @pl.when(f == 0)
    def _gather():
        tok = tok_ref[0, 0, :]
        io = jax.lax.broadcasted_iota(jnp.int32, (bt, t), 1)
        g = (io == tok[:, None]).astype(jnp.bfloat16)
        xg_ref[...] = jnp.dot(
            g, x_ref[...],
            preferred_element_type=jnp.float32).astype(jnp.bfloat16)

    xg = xg_ref[...]
    w1b = w1_ref[0].astype(jnp.bfloat16)
    w3b = w3_ref[0].astype(jnp.bfloat16)
    w2b = w2_ref[0].astype(jnp.bfloat16)
    a = jax.lax.dot_general(xg, w1b, (((1,), (1,)), ((), ())),
                            preferred_element_type=jnp.float32)
    c = jax.lax.dot_general(xg, w3b, (((1,), (1,)), ((), ())),
                            preferred_element_type=jnp.float32)
    h = a * jax.nn.sigmoid(a) * c
    part = jax.lax.dot_general(h.astype(jnp.bfloat16), w2b,
                               (((1,), (1,)), ((), ())),
                               preferred_element_type=jnp.float32)

    @pl.when(f == 0)
    def _set():
        y_ref[...] = part

    @pl.when(f > 0)
    def _acc():
        y_ref[...] += part

    @pl.when(f == nf - 1)
    def _scatter():
        tok = tok_ref[0, 0, :]
        wv = pw_ref[0, 0, :]
        io = jax.lax.broadcasted_iota(jnp.int32, (bt, t), 1)
        s = jnp.where(io == tok[:, None], wv[:, None],
                      0.0).astype(jnp.bfloat16)
        out_ref[...] += jax.lax.dot_general(
            s, y_ref[...].astype(jnp.bfloat16), (((0,), (0,)), ((), ())),
            preferred_element_type=jnp.float32)


def _grouped_ffn(x, w1, w3, w2, block_expert, ptok, pw, bt, bf):
    t, d = x.shape
    e, f_dim, _ = w1.shape
    nb = ptok.shape[0] // bt
    nf = f_dim // bf
    tok3 = ptok.reshape(nb, 1, bt)
    pw3 = pw.reshape(nb, 1, bt)
    grid_spec = pltpu.PrefetchScalarGridSpec(
        num_scalar_prefetch=1,
        grid=(nb, nf),
        in_specs=[
            pl.BlockSpec((t, d), lambda b, f, be: (0, 0)),
            pl.BlockSpec((1, bf, d), lambda b, f, be: (be[b], f, 0)),
            pl.BlockSpec((1, bf, d), lambda b, f, be: (be[b], f, 0)),
            pl.BlockSpec((1, d, bf), lambda b, f, be: (be[b], 0, f)),
            pl.BlockSpec((1, 1, bt), lambda b, f, be: (b, 0, 0)),
            pl.BlockSpec((1, 1, bt), lambda b, f, be: (b, 0, 0)),
        ],
        out_specs=pl.BlockSpec((t, d), lambda b, f, be: (0, 0)),
        scratch_shapes=[
            pltpu.VMEM((bt, d), jnp.bfloat16),
            pltpu.VMEM((bt, d), jnp.float32),
        ],
    )
    return pl.pallas_call(
        functools.partial(_ffn_body, nf, t),
        grid_spec=grid_spec,
        out_shape=jax.ShapeDtypeStruct((t, d), jnp.float32),
    )(block_expert, x, w1, w3, w2, tok3, pw3)


def _routing_metadata(topi, topw, e, bt, nb):
    a = topi.size
    n_pad = nb * bt
    flat_e = topi.reshape(-1).astype(jnp.int32)
    order = jnp.argsort(flat_e, stable=True)
    e_sorted = flat_e[order]
    tok_sorted = (order // TOP_K).astype(jnp.int32)
    w_sorted = topw.reshape(-1)[order]
    counts = jnp.zeros((e,), jnp.int32).at[flat_e].add(1)
    nblk_e = (counts + bt - 1) // bt
    pad_start = (jnp.cumsum(nblk_e) - nblk_e) * bt
    seg_start = jnp.cumsum(counts) - counts
    rank = jnp.arange(a, dtype=jnp.int32) - seg_start[e_sorted]
    pos = pad_start[e_sorted] + rank
    ptok = jnp.zeros((n_pad,), jnp.int32).at[pos].set(tok_sorted)
    pw = jnp.zeros((n_pad,), jnp.float32).at[pos].set(w_sorted)
    pad_end_blocks = jnp.cumsum(nblk_e)
    bid = jnp.arange(nb, dtype=jnp.int32)
    block_expert = jnp.sum(
        (bid[:, None] >= pad_end_blocks[None, :]).astype(jnp.int32), axis=1)
    block_expert = jnp.minimum(block_expert, e - 1).astype(jnp.int32)
    return block_expert, ptok, pw


def kernel(hidden_states, gate_w, w1, w3, w2):
    input_shape = hidden_states.shape
    d = input_shape[-1]
    t = hidden_states.size // d
    e, f_dim, _ = w1.shape
    bt = 384
    bf = 256
    nb = (t * TOP_K) // bt + e
    x = hidden_states.reshape(t, d)
    topw, topi = _router(x, gate_w)
    block_expert, ptok, pw = _routing_metadata(topi, topw, e, bt, nb)
    out = _grouped_ffn(x.astype(jnp.bfloat16), w1, w3, w2,
                       block_expert, ptok, pw, bt, bf)
    return out.reshape(input_shape)


# speedup vs baseline: 1.0534x; 1.0534x over previous
import functools

import jax
import jax.numpy as jnp
from jax.experimental import pallas as pl
from jax.experimental.pallas import tpu as pltpu

TOP_K = 2
LANES = 128


def _router_body(nexp, x_ref, g_ref, wout_ref, iout_ref):
    bt = x_ref.shape[0]
    logits = jax.lax.dot_general(
        x_ref[...], g_ref[...], (((1,), (1,)), ((), ())),
        preferred_element_type=jnp.float32)
    io = jax.lax.broadcasted_iota(jnp.int32, (bt, LANES), 1)
    valid = io < nexp
    logits = jnp.where(valid, logits, -jnp.inf)
    m = jnp.max(logits, axis=1, keepdims=True)
    p = jnp.exp(logits - m)
    p = p / jnp.sum(p, axis=1, keepdims=True)
    m1 = jnp.max(p, axis=1, keepdims=True)
    i1 = jnp.min(jnp.where(p == m1, io, LANES), axis=1, keepdims=True)
    p2 = jnp.where(io == i1, -1.0, p)
    m2 = jnp.max(p2, axis=1, keepdims=True)
    i2 = jnp.min(jnp.where(p2 == m2, io, LANES), axis=1, keepdims=True)
    s = m1 + m2
    wa = m1 / s
    wb = m2 / s
    wout_ref[...] = jnp.where(io == 0, wa, jnp.where(io == 1, wb, 0.0))
    iout_ref[...] = jnp.where(io == 0, i1, jnp.where(io == 1, i2, 0))


def _router(x, gate_w, bt=256):
    t, d = x.shape
    e = gate_w.shape[0]
    gate_pad = jnp.zeros((LANES, d), jnp.float32).at[:e].set(gate_w)
    wout, iout = pl.pallas_call(
        functools.partial(_router_body, e),
        grid=(t // bt,),
        in_specs=[
            pl.BlockSpec((bt, d), lambda i: (i, 0)),
            pl.BlockSpec((LANES, d), lambda i: (0, 0)),
        ],
        out_specs=[
            pl.BlockSpec((bt, LANES), lambda i: (i, 0)),
            pl.BlockSpec((bt, LANES), lambda i: (i, 0)),
        ],
        out_shape=[
            jax.ShapeDtypeStruct((t, LANES), jnp.float32),
            jax.ShapeDtypeStruct((t, LANES), jnp.int32),
        ],
    )(x, gate_pad)
    return wout[:, :TOP_K], iout[:, :TOP_K]


def _ffn_body(nf, t, be_ref, x_ref, w1_ref, w3_ref, w2_ref, tok_ref, pw_ref,
              out_ref, xg_ref, y_ref):
    b = pl.program_id(0)
    f = pl.program_id(1)
    bt = xg_ref.shape[0]

    @pl.when(jnp.logical_and(b == 0, f == 0))
    def _init():
        out_ref[...] = jnp.zeros_like(out_ref)

    @pl.when(f == 0)
    def _gather():
        tok = tok_ref[0, 0, :]
        io = jax.lax.broadcasted_iota(jnp.int32, (bt, t), 1)
        g = (io == tok[:, None]).astype(jnp.bfloat16)
        xg_ref[...] = jnp.dot(
            g, x_ref[...],
            preferred_element_type=jnp.float32).astype(jnp.bfloat16)

    xg = xg_ref[...].astype(jnp.float32)
    a = jax.lax.dot_general(xg, w1_ref[0], (((1,), (1,)), ((), ())),
                            preferred_element_type=jnp.float32)
    c = jax.lax.dot_general(xg, w3_ref[0], (((1,), (1,)), ((), ())),
                            preferred_element_type=jnp.float32)
    h = a * jax.nn.sigmoid(a) * c
    part = jax.lax.dot_general(h, w2_ref[0], (((1,), (1,)), ((), ())),
                               preferred_element_type=jnp.float32)

    @pl.when(f == 0)
    def _set():
        y_ref[...] = part

    @pl.when(f > 0)
    def _acc():
        y_ref[...] += part

    @pl.when(f == nf - 1)
    def _scatter():
        tok = tok_ref[0, 0, :]
        wv = pw_ref[0, 0, :]
        io = jax.lax.broadcasted_iota(jnp.int32, (bt, t), 1)
        s = jnp.where(io == tok[:, None], wv[:, None],
                      0.0).astype(jnp.bfloat16)
        out_ref[...] += jax.lax.dot_general(
            s, y_ref[...].astype(jnp.bfloat16), (((0,), (0,)), ((), ())),
            preferred_element_type=jnp.float32)


def _grouped_ffn(x, w1, w3, w2, block_expert, ptok, pw, bt, bf):
    t, d = x.shape
    e, f_dim, _ = w1.shape
    nb = ptok.shape[0] // bt
    nf = f_dim // bf
    tok3 = ptok.reshape(nb, 1, bt)
    pw3 = pw.reshape(nb, 1, bt)
    grid_spec = pltpu.PrefetchScalarGridSpec(
        num_scalar_prefetch=1,
        grid=(nb, nf),
        in_specs=[
            pl.BlockSpec((t, d), lambda b, f, be: (0, 0)),
            pl.BlockSpec((1, bf, d), lambda b, f, be: (be[b], f, 0)),
            pl.BlockSpec((1, bf, d), lambda b, f, be: (be[b], f, 0)),
            pl.BlockSpec((1, d, bf), lambda b, f, be: (be[b], 0, f)),
            pl.BlockSpec((1, 1, bt), lambda b, f, be: (b, 0, 0)),
            pl.BlockSpec((1, 1, bt), lambda b, f, be: (b, 0, 0)),
        ],
        out_specs=pl.BlockSpec((t, d), lambda b, f, be: (0, 0)),
        scratch_shapes=[
            pltpu.VMEM((bt, d), jnp.bfloat16),
            pltpu.VMEM((bt, d), jnp.float32),
        ],
    )
    return pl.pallas_call(
        functools.partial(_ffn_body, nf, t),
        grid_spec=grid_spec,
        out_shape=jax.ShapeDtypeStruct((t, d), jnp.float32),
    )(block_expert, x, w1, w3, w2, tok3, pw3)


def _routing_metadata(topi, topw, e, bt, nb):
    a = topi.size
    n_pad = nb * bt
    flat_e = topi.reshape(-1).astype(jnp.int32)
    order = jnp.argsort(flat_e, stable=True)
    e_sorted = flat_e[order]
    tok_sorted = (order // TOP_K).astype(jnp.int32)
    w_sorted = topw.reshape(-1)[order]
    counts = jnp.zeros((e,), jnp.int32).at[flat_e].add(1)
    nblk_e = (counts + bt - 1) // bt
    pad_start = (jnp.cumsum(nblk_e) - nblk_e) * bt
    seg_start = jnp.cumsum(counts) - counts
    rank = jnp.arange(a, dtype=jnp.int32) - seg_start[e_sorted]
    pos = pad_start[e_sorted] + rank
    ptok = jnp.zeros((n_pad,), jnp.int32).at[pos].set(tok_sorted)
    pw = jnp.zeros((n_pad,), jnp.float32).at[pos].set(w_sorted)
    pad_end_blocks = jnp.cumsum(nblk_e)
    bid = jnp.arange(nb, dtype=jnp.int32)
    block_expert = jnp.sum(
        (bid[:, None] >= pad_end_blocks[None, :]).astype(jnp.int32), axis=1)
    block_expert = jnp.minimum(block_expert, e - 1).astype(jnp.int32)
    return block_expert, ptok, pw


def kernel(hidden_states, gate_w, w1, w3, w2):
    input_shape = hidden_states.shape
    d = input_shape[-1]
    t = hidden_states.size // d
    e, f_dim, _ = w1.shape
    bt = 256
    bf = 512
    nb = (t * TOP_K) // bt + e
    x = hidden_states.reshape(t, d)
    topw, topi = _router(x, gate_w)
    block_expert, ptok, pw = _routing_metadata(topi, topw, e, bt, nb)
    out = _grouped_ffn(x.astype(jnp.bfloat16), w1, w3, w2,
                       block_expert, ptok, pw, bt, bf)
    return out.reshape(input_shape)


# speedup vs baseline: 1.1402x; 1.0824x over previous
import functools

import jax
import jax.numpy as jnp
from jax.experimental import pallas as pl
from jax.experimental.pallas import tpu as pltpu

TOP_K = 2
LANES = 128


def _router_body(nexp, x_ref, g_ref, wout_ref, iout_ref):
    bt = x_ref.shape[0]
    logits = jax.lax.dot_general(
        x_ref[...], g_ref[...], (((1,), (1,)), ((), ())),
        preferred_element_type=jnp.float32)
    io = jax.lax.broadcasted_iota(jnp.int32, (bt, LANES), 1)
    valid = io < nexp
    logits = jnp.where(valid, logits, -jnp.inf)
    m = jnp.max(logits, axis=1, keepdims=True)
    p = jnp.exp(logits - m)
    p = p / jnp.sum(p, axis=1, keepdims=True)
    m1 = jnp.max(p, axis=1, keepdims=True)
    i1 = jnp.min(jnp.where(p == m1, io, LANES), axis=1, keepdims=True)
    p2 = jnp.where(io == i1, -1.0, p)
    m2 = jnp.max(p2, axis=1, keepdims=True)
    i2 = jnp.min(jnp.where(p2 == m2, io, LANES), axis=1, keepdims=True)
    s = m1 + m2
    wa = m1 / s
    wb = m2 / s
    wout_ref[...] = jnp.where(io == 0, wa, jnp.where(io == 1, wb, 0.0))
    iout_ref[...] = jnp.where(io == 0, i1, jnp.where(io == 1, i2, 0))


def _router(x, gate_w, bt=256):
    t, d = x.shape
    e = gate_w.shape[0]
    gate_pad = jnp.zeros((LANES, d), jnp.float32).at[:e].set(gate_w)
    wout, iout = pl.pallas_call(
        functools.partial(_router_body, e),
        grid=(t // bt,),
        in_specs=[
            pl.BlockSpec((bt, d), lambda i: (i, 0)),
            pl.BlockSpec((LANES, d), lambda i: (0, 0)),
        ],
        out_specs=[
            pl.BlockSpec((bt, LANES), lambda i: (i, 0)),
            pl.BlockSpec((bt, LANES), lambda i: (i, 0)),
        ],
        out_shape=[
            jax.ShapeDtypeStruct((t, LANES), jnp.float32),
            jax.ShapeDtypeStruct((t, LANES), jnp.int32),
        ],
    )(x, gate_pad)
    return wout[:, :TOP_K], iout[:, :TOP_K]


def _ffn_body(nf, t, be_ref, x_ref, w1_ref, w3_ref, w2_ref, tok_ref, pw_ref,
              out_ref, xg_ref, y_ref):
    b = pl.program_id(0)
    f = pl.program_id(1)
    bt = xg_ref.shape[0]

    @pl.when(jnp.logical_and(b == 0, f == 0))
    def _init():
        out_ref[...] = jnp.zeros_like(out_ref)

    @pl.when(f == 0)
    def _gather():
        tok = tok_ref[0, 0, :]
        io = jax.lax.broadcasted_iota(jnp.int32, (bt, t), 1)
        g = (io == tok[:, None]).astype(jnp.bfloat16)
        xg_ref[...] = jnp.dot(
            g, x_ref[...],
            preferred_element_type=jnp.float32).astype(jnp.bfloat16)

    xg = xg_ref[...].astype(jnp.float32)
    a = jax.lax.dot_general(xg, w1_ref[0], (((1,), (1,)), ((), ())),
                            preferred_element_type=jnp.float32)
    c = jax.lax.dot_general(xg, w3_ref[0], (((1,), (1,)), ((), ())),
                            preferred_element_type=jnp.float32)
    h = a * jax.nn.sigmoid(a) * c
    part = jax.lax.dot_general(h, w2_ref[0], (((1,), (1,)), ((), ())),
                               preferred_element_type=jnp.float32)

    @pl.when(f == 0)
    def _set():
        y_ref[...] = part

    @pl.when(f > 0)
    def _acc():
        y_ref[...] += part

    @pl.when(f == nf - 1)
    def _scatter():
        tok = tok_ref[0, 0, :]
        wv = pw_ref[0, 0, :]
        io = jax.lax.broadcasted_iota(jnp.int32, (bt, t), 1)
        s = jnp.where(io == tok[:, None], wv[:, None],
                      0.0).astype(jnp.bfloat16)
        delta = jax.lax.dot_general(
            s, y_ref[...].astype(jnp.bfloat16), (((0,), (0,)), ((), ())),
            preferred_element_type=jnp.float32)
        out_ref[...] = (out_ref[...].astype(jnp.float32)
                        + delta).astype(jnp.bfloat16)


def _grouped_ffn(x, w1, w3, w2, block_expert, ptok, pw, bt, bf):
    t, d = x.shape
    e, f_dim, _ = w1.shape
    nb = ptok.shape[0] // bt
    nf = f_dim // bf
    tok3 = ptok.reshape(nb, 1, bt)
    pw3 = pw.reshape(nb, 1, bt)
    grid_spec = pltpu.PrefetchScalarGridSpec(
        num_scalar_prefetch=1,
        grid=(nb, nf),
        in_specs=[
            pl.BlockSpec((t, d), lambda b, f, be: (0, 0)),
            pl.BlockSpec((1, bf, d), lambda b, f, be: (be[b], f, 0)),
            pl.BlockSpec((1, bf, d), lambda b, f, be: (be[b], f, 0)),
            pl.BlockSpec((1, d, bf), lambda b, f, be: (be[b], 0, f)),
            pl.BlockSpec((1, 1, bt), lambda b, f, be: (b, 0, 0)),
            pl.BlockSpec((1, 1, bt), lambda b, f, be: (b, 0, 0)),
        ],
        out_specs=pl.BlockSpec((t, d), lambda b, f, be: (0, 0)),
        scratch_shapes=[
            pltpu.VMEM((bt, d), jnp.bfloat16),
            pltpu.VMEM((bt, d), jnp.float32),
        ],
    )
    return pl.pallas_call(
        functools.partial(_ffn_body, nf, t),
        grid_spec=grid_spec,
        out_shape=jax.ShapeDtypeStruct((t, d), jnp.bfloat16),
    )(block_expert, x, w1, w3, w2, tok3, pw3)


def _routing_metadata(topi, topw, e, bt, nb):
    a = topi.size
    n_pad = nb * bt
    flat_e = topi.reshape(-1).astype(jnp.int32)
    order = jnp.argsort(flat_e, stable=True)
    e_sorted = flat_e[order]
    tok_sorted = (order // TOP_K).astype(jnp.int32)
    w_sorted = topw.reshape(-1)[order]
    counts = jnp.zeros((e,), jnp.int32).at[flat_e].add(1)
    nblk_e = (counts + bt - 1) // bt
    pad_start = (jnp.cumsum(nblk_e) - nblk_e) * bt
    seg_start = jnp.cumsum(counts) - counts
    rank = jnp.arange(a, dtype=jnp.int32) - seg_start[e_sorted]
    pos = pad_start[e_sorted] + rank
    ptok = jnp.zeros((n_pad,), jnp.int32).at[pos].set(tok_sorted)
    pw = jnp.zeros((n_pad,), jnp.float32).at[pos].set(w_sorted)
    pad_end_blocks = jnp.cumsum(nblk_e)
    bid = jnp.arange(nb, dtype=jnp.int32)
    block_expert = jnp.sum(
        (bid[:, None] >= pad_end_blocks[None, :]).astype(jnp.int32), axis=1)
    block_expert = jnp.minimum(block_expert, e - 1).astype(jnp.int32)
    return block_expert, ptok, pw


def kernel(hidden_states, gate_w, w1, w3, w2):
    input_shape = hidden_states.shape
    d = input_shape[-1]
    t = hidden_states.size // d
    e, f_dim, _ = w1.shape
    bt = 384
    bf = 512
    nb = (t * TOP_K) // bt + e
    x = hidden_states.reshape(t, d)
    topw, topi = _router(x, gate_w)
    block_expert, ptok, pw = _routing_metadata(topi, topw, e, bt, nb)
    out = _grouped_ffn(x.astype(jnp.bfloat16), w1, w3, w2,
                       block_expert, ptok, pw, bt, bf)
    return out.astype(jnp.float32).reshape(input_shape)
